# baseline (speedup 1.0000x reference)
import functools

import jax
import jax.numpy as jnp
from jax import lax
from jax.experimental import pallas as pl
from jax.experimental.pallas import tpu as pltpu
from jax.experimental.pallas import tpu_sc as plsc

_G = 64
_NC = 2
_NS = 16
_CHUNK = 40
_NSLOT = 8


def _make_segsum(n, d, e):
    epc = e // _NC
    ept = epc // _NS
    ncpt = ept // _CHUNK
    niter = ncpt // _NSLOT
    ntail = ncpt - niter * _NSLOT
    assert ntail < _NSLOT
    stripe_stride = 8 * (n // (8 * _NS))
    stripe_rows = n - (_NS - 1) * stripe_stride
    mesh = plsc.VectorSubcoreMesh(core_axis_name="c", subcore_axis_name="s")

    @functools.partial(
        pl.kernel,
        out_type=jax.ShapeDtypeStruct((_NC, n, d), jnp.float32),
        mesh=mesh,
        scratch_types=[
            pltpu.VMEM((2, _NSLOT, _CHUNK), jnp.int32),
            pltpu.VMEM((2, _NSLOT, _CHUNK), jnp.int32),
            pltpu.VMEM((_NSLOT, _CHUNK, d), jnp.float32),
            pltpu.VMEM_SHARED((n, d), jnp.float32),
            [pltpu.SemaphoreType.DMA] * _NSLOT,
            [pltpu.SemaphoreType.DMA] * _NSLOT,
            [pltpu.SemaphoreType.DMA] * _NSLOT,
        ],
    )
    def segsum(x_hbm, src_hbm, dst_hbm, zeros_hbm, out_hbm,
               src_v, dst_v, rows_v, agg_sh, isem, gsem, ssem):
        c = lax.axis_index("c")
        s = lax.axis_index("s")
        pltpu.sync_copy(zeros_hbm.at[pl.ds(s * stripe_stride, stripe_rows)],
                        agg_sh.at[pl.ds(s * stripe_stride, stripe_rows)])
        plsc.subcore_barrier()
        base = c * epc + s * ept

        def start_idx(i, b, k):
            off = base + i * _CHUNK
            pltpu.async_copy(src_hbm.at[pl.ds(off, _CHUNK)],
                             src_v.at[b, k], isem[k])
            pltpu.async_copy(dst_hbm.at[pl.ds(off, _CHUNK)],
                             dst_v.at[b, k], isem[k])

        def wait_idx(i, b, k):
            off = base + i * _CHUNK
            pltpu.make_async_copy(src_hbm.at[pl.ds(off, _CHUNK)],
                                  src_v.at[b, k], isem[k]).wait()
            pltpu.make_async_copy(dst_hbm.at[pl.ds(off, _CHUNK)],
                                  dst_v.at[b, k], isem[k]).wait()

        def start_g(b, k):
            pltpu.async_copy(x_hbm.at[src_v.at[b, k]], rows_v.at[k], gsem[k])

        def wait_g(b, k):
            pltpu.make_async_copy(x_hbm.at[src_v.at[b, k]], rows_v.at[k],
                                  gsem[k]).wait()

        def start_s(b, k):
            pltpu.async_copy(rows_v.at[k], agg_sh.at[dst_v.at[b, k]],
                             ssem[k], add=True)

        def wait_s(b, k):
            pltpu.make_async_copy(rows_v.at[k], agg_sh.at[dst_v.at[b, k]],
                                  ssem[k]).wait()

        for k in range(_NSLOT):
            start_idx(k, 0, k)

        def body(j, carry):
            b = lax.rem(j, 2)
            nb = lax.rem(j + 1, 2)
            i0 = j * _NSLOT

            @pl.when(j > 0)
            def _():
                for k in range(_NSLOT):
                    wait_s(nb, k)

            for k in range(_NSLOT):
                wait_idx(i0 + k, b, k)
                start_g(b, k)

            @pl.when(j < niter - 1)
            def _():
                for k in range(_NSLOT):
                    start_idx(i0 + _NSLOT + k, nb, k)

            for k in range(_NSLOT):
                wait_g(b, k)
                start_s(b, k)
            return carry

        lax.fori_loop(0, niter, body, 0)
        last_b = (niter - 1) % 2
        for k in range(_NSLOT):
            wait_s(last_b, k)
        for k in range(ntail):
            i = niter * _NSLOT + k
            off = base + i * _CHUNK
            pltpu.sync_copy(src_hbm.at[pl.ds(off, _CHUNK)],
                            src_v.at[last_b, k])
            pltpu.sync_copy(dst_hbm.at[pl.ds(off, _CHUNK)],
                            dst_v.at[last_b, k])
            start_g(last_b, k)
        for k in range(ntail):
            wait_g(last_b, k)
            start_s(last_b, k)
        for k in range(ntail):
            wait_s(last_b, k)
        plsc.subcore_barrier()
        pltpu.sync_copy(agg_sh.at[pl.ds(s * stripe_stride, stripe_rows)],
                        out_hbm.at[c].at[pl.ds(s * stripe_stride, stripe_rows)])

    return segsum


def _bn_relu(h, g_row, bb_row):
    m = jnp.mean(h, axis=0, keepdims=True)
    v = jnp.mean((h - m) * (h - m), axis=0, keepdims=True)
    return jnp.maximum((h - m) * lax.rsqrt(v + 1e-5) * g_row + bb_row, 0.0)


def _lin0_body(x_ref, w_ref, b_ref, g_ref, bb_ref, o_ref):
    h = jnp.dot(x_ref[...], w_ref[...], preferred_element_type=jnp.float32)
    o_ref[...] = _bn_relu(h + b_ref[...], g_ref[...], bb_ref[...])


def _gin_mlp(z, w1_ref, b1_ref, g_ref, bb_ref, w2_ref, b2_ref):
    h = jnp.dot(z, w1_ref[...], preferred_element_type=jnp.float32) + b1_ref[...]
    h = _bn_relu(h, g_ref[...], bb_ref[...])
    h = jnp.dot(h, w2_ref[...], preferred_element_type=jnp.float32) + b2_ref[...]
    return jnp.maximum(h, 0.0)


def _pool(batch_ref, ho, n):
    onehot = (lax.broadcasted_iota(jnp.int32, (_G, n), 0)
              == batch_ref[...]).astype(jnp.float32)
    return jnp.dot(onehot, ho, preferred_element_type=jnp.float32)


def _layer_body(x_ref, parts_ref, w1_ref, b1_ref, g_ref, bb_ref, w2_ref,
                b2_ref, batch_ref, ho_ref, pool_ref):
    z = x_ref[...] + parts_ref[0] + parts_ref[1]
    ho = _gin_mlp(z, w1_ref, b1_ref, g_ref, bb_ref, w2_ref, b2_ref)
    ho_ref[...] = ho
    pool_ref[...] = _pool(batch_ref, ho, x_ref.shape[0])


def _layer3_head_body(x_ref, parts_ref, w1_ref, b1_ref, g_ref, bb_ref,
                      w2_ref, b2_ref, batch_ref, p1_ref, p2_ref,
                      lin1w_ref, lin1b_ref, lin2w_ref, lin2b_ref, out_ref):
    z = x_ref[...] + parts_ref[0] + parts_ref[1]
    ho = _gin_mlp(z, w1_ref, b1_ref, g_ref, bb_ref, w2_ref, b2_ref)
    p3 = _pool(batch_ref, ho, x_ref.shape[0])
    p = jnp.concatenate([p1_ref[...], p2_ref[...], p3], axis=1)
    hh = jnp.dot(p, lin1w_ref[...], preferred_element_type=jnp.float32)
    hh = jnp.maximum(hh + lin1b_ref[...], 0.0)
    out_ref[...] = (jnp.dot(hh, lin2w_ref[...],
                            preferred_element_type=jnp.float32)
                    + lin2b_ref[...])


def kernel(x, edge_index, batch, W_lin, b_lin, bn0_g, bn0_b,
           c1_W1, c1_b1, c1_g, c1_bb, c1_W2, c1_b2,
           c2_W1, c2_b1, c2_g, c2_bb, c2_W2, c2_b2,
           c3_W1, c3_b1, c3_g, c3_bb, c3_W2, c3_b2,
           lin1_W, lin1_b, lin2_W, lin2_b):
    n, d = x.shape
    e = edge_index.shape[1]
    c = lin2_W.shape[1]
    x = x.astype(jnp.float32)
    src = edge_index[0]
    dst = edge_index[1]
    batch_row = batch.reshape(1, n)
    zeros = jnp.zeros((n, d), jnp.float32)
    row = lambda a: a.reshape(1, -1)

    x0 = pl.pallas_call(
        _lin0_body,
        out_shape=jax.ShapeDtypeStruct((n, d), jnp.float32),
    )(x, W_lin, row(b_lin), row(bn0_g), row(bn0_b))

    segsum = _make_segsum(n, d, e)

    layer_call = pl.pallas_call(
        _layer_body,
        out_shape=(jax.ShapeDtypeStruct((n, d), jnp.float32),
                   jax.ShapeDtypeStruct((_G, d), jnp.float32)),
    )

    parts1 = segsum(x0, src, dst, zeros)
    h1, p1 = layer_call(x0, parts1, c1_W1, row(c1_b1), row(c1_g), row(c1_bb),
                        c1_W2, row(c1_b2), batch_row)
    parts2 = segsum(h1, src, dst, zeros)
    h2, p2 = layer_call(h1, parts2, c2_W1, row(c2_b1), row(c2_g), row(c2_bb),
                        c2_W2, row(c2_b2), batch_row)
    parts3 = segsum(h2, src, dst, zeros)
    out = pl.pallas_call(
        _layer3_head_body,
        out_shape=jax.ShapeDtypeStruct((_G, c), jnp.float32),
    )(h2, parts3, c3_W1, row(c3_b1), row(c3_g), row(c3_bb), c3_W2,
      row(c3_b2), batch_row, p1, p2, lin1_W, row(lin1_b), lin2_W,
      row(lin2_b))
    return out


# speedup vs baseline: 1.0307x; 1.0307x over previous
import functools

import jax
import jax.numpy as jnp
from jax import lax
from jax.experimental import pallas as pl
from jax.experimental.pallas import tpu as pltpu
from jax.experimental.pallas import tpu_sc as plsc

_G = 64
_NC = 2
_NS = 16
_CHUNK = 40
_NSLOT = 7


def _make_segsum(n, d, e):
    epc = e // _NC
    ept = epc // _NS
    ncpt = ept // _CHUNK
    niter = ncpt // _NSLOT
    ntail = ncpt - niter * _NSLOT
    assert ntail < _NSLOT
    stripe_stride = 8 * (n // (8 * _NS))
    stripe_rows = n - (_NS - 1) * stripe_stride
    mesh = plsc.VectorSubcoreMesh(core_axis_name="c", subcore_axis_name="s")

    @functools.partial(
        pl.kernel,
        out_type=jax.ShapeDtypeStruct((_NC, n, d), jnp.float32),
        mesh=mesh,
        scratch_types=[
            pltpu.VMEM((2, _NSLOT, _CHUNK), jnp.int32),
            pltpu.VMEM((2, _NSLOT, _CHUNK), jnp.int32),
            pltpu.VMEM((_NSLOT, _CHUNK, d), jnp.float32),
            pltpu.VMEM_SHARED((n, d), jnp.float32),
            [pltpu.SemaphoreType.DMA] * _NSLOT,
            [pltpu.SemaphoreType.DMA] * _NSLOT,
            [pltpu.SemaphoreType.DMA] * _NSLOT,
        ],
    )
    def segsum(x_hbm, src_hbm, dst_hbm, zeros_hbm, out_hbm,
               src_v, dst_v, rows_v, agg_sh, isem, gsem, ssem):
        c = lax.axis_index("c")
        s = lax.axis_index("s")
        pltpu.sync_copy(zeros_hbm.at[pl.ds(s * stripe_stride, stripe_rows)],
                        agg_sh.at[pl.ds(s * stripe_stride, stripe_rows)])
        plsc.subcore_barrier()
        base = c * epc + s * ept

        def start_idx(i, b, k):
            off = base + i * _CHUNK
            pltpu.async_copy(src_hbm.at[pl.ds(off, _CHUNK)],
                             src_v.at[b, k], isem[k])
            pltpu.async_copy(dst_hbm.at[pl.ds(off, _CHUNK)],
                             dst_v.at[b, k], isem[k])

        def wait_idx(i, b, k):
            off = base + i * _CHUNK
            pltpu.make_async_copy(src_hbm.at[pl.ds(off, _CHUNK)],
                                  src_v.at[b, k], isem[k]).wait()
            pltpu.make_async_copy(dst_hbm.at[pl.ds(off, _CHUNK)],
                                  dst_v.at[b, k], isem[k]).wait()

        def start_g(b, k):
            pltpu.async_copy(x_hbm.at[src_v.at[b, k]], rows_v.at[k], gsem[k])

        def wait_g(b, k):
            pltpu.make_async_copy(x_hbm.at[src_v.at[b, k]], rows_v.at[k],
                                  gsem[k]).wait()

        def start_s(b, k):
            pltpu.async_copy(rows_v.at[k], agg_sh.at[dst_v.at[b, k]],
                             ssem[k], add=True)

        def wait_s(b, k):
            pltpu.make_async_copy(rows_v.at[k], agg_sh.at[dst_v.at[b, k]],
                                  ssem[k]).wait()

        for k in range(_NSLOT):
            start_idx(k, 0, k)

        def body(j, carry):
            b = lax.rem(j, 2)
            nb = lax.rem(j + 1, 2)
            i0 = j * _NSLOT

            @pl.when(j > 0)
            def _():
                for k in range(_NSLOT):
                    wait_s(nb, k)

            for k in range(_NSLOT):
                wait_idx(i0 + k, b, k)
                start_g(b, k)

            @pl.when(j < niter - 1)
            def _():
                for k in range(_NSLOT):
                    start_idx(i0 + _NSLOT + k, nb, k)

            for k in range(_NSLOT):
                wait_g(b, k)
                start_s(b, k)
            return carry

        lax.fori_loop(0, niter, body, 0)
        last_b = (niter - 1) % 2
        for k in range(_NSLOT):
            wait_s(last_b, k)
        for k in range(ntail):
            i = niter * _NSLOT + k
            off = base + i * _CHUNK
            pltpu.sync_copy(src_hbm.at[pl.ds(off, _CHUNK)],
                            src_v.at[last_b, k])
            pltpu.sync_copy(dst_hbm.at[pl.ds(off, _CHUNK)],
                            dst_v.at[last_b, k])
            start_g(last_b, k)
        for k in range(ntail):
            wait_g(last_b, k)
            start_s(last_b, k)
        for k in range(ntail):
            wait_s(last_b, k)
        plsc.subcore_barrier()
        pltpu.sync_copy(agg_sh.at[pl.ds(s * stripe_stride, stripe_rows)],
                        out_hbm.at[c].at[pl.ds(s * stripe_stride, stripe_rows)])

    return segsum


def _bn_relu(h, g_row, bb_row):
    m = jnp.mean(h, axis=0, keepdims=True)
    v = jnp.mean((h - m) * (h - m), axis=0, keepdims=True)
    return jnp.maximum((h - m) * lax.rsqrt(v + 1e-5) * g_row + bb_row, 0.0)


def _lin0_body(x_ref, w_ref, b_ref, g_ref, bb_ref, o_ref):
    h = jnp.dot(x_ref[...], w_ref[...], preferred_element_type=jnp.float32)
    o_ref[...] = _bn_relu(h + b_ref[...], g_ref[...], bb_ref[...])


def _gin_mlp(z, w1_ref, b1_ref, g_ref, bb_ref, w2_ref, b2_ref):
    h = jnp.dot(z, w1_ref[...], preferred_element_type=jnp.float32) + b1_ref[...]
    h = _bn_relu(h, g_ref[...], bb_ref[...])
    h = jnp.dot(h, w2_ref[...], preferred_element_type=jnp.float32) + b2_ref[...]
    return jnp.maximum(h, 0.0)


def _pool(batch_ref, ho, n):
    onehot = (lax.broadcasted_iota(jnp.int32, (_G, n), 0)
              == batch_ref[...]).astype(jnp.float32)
    return jnp.dot(onehot, ho, preferred_element_type=jnp.float32)


def _layer_body(x_ref, parts_ref, w1_ref, b1_ref, g_ref, bb_ref, w2_ref,
                b2_ref, batch_ref, ho_ref, pool_ref):
    z = x_ref[...] + parts_ref[0] + parts_ref[1]
    ho = _gin_mlp(z, w1_ref, b1_ref, g_ref, bb_ref, w2_ref, b2_ref)
    ho_ref[...] = ho
    pool_ref[...] = _pool(batch_ref, ho, x_ref.shape[0])


def _layer3_head_body(x_ref, parts_ref, w1_ref, b1_ref, g_ref, bb_ref,
                      w2_ref, b2_ref, batch_ref, p1_ref, p2_ref,
                      lin1w_ref, lin1b_ref, lin2w_ref, lin2b_ref, out_ref):
    z = x_ref[...] + parts_ref[0] + parts_ref[1]
    ho = _gin_mlp(z, w1_ref, b1_ref, g_ref, bb_ref, w2_ref, b2_ref)
    p3 = _pool(batch_ref, ho, x_ref.shape[0])
    p = jnp.concatenate([p1_ref[...], p2_ref[...], p3], axis=1)
    hh = jnp.dot(p, lin1w_ref[...], preferred_element_type=jnp.float32)
    hh = jnp.maximum(hh + lin1b_ref[...], 0.0)
    out_ref[...] = (jnp.dot(hh, lin2w_ref[...],
                            preferred_element_type=jnp.float32)
                    + lin2b_ref[...])


def kernel(x, edge_index, batch, W_lin, b_lin, bn0_g, bn0_b,
           c1_W1, c1_b1, c1_g, c1_bb, c1_W2, c1_b2,
           c2_W1, c2_b1, c2_g, c2_bb, c2_W2, c2_b2,
           c3_W1, c3_b1, c3_g, c3_bb, c3_W2, c3_b2,
           lin1_W, lin1_b, lin2_W, lin2_b):
    n, d = x.shape
    e = edge_index.shape[1]
    c = lin2_W.shape[1]
    x = x.astype(jnp.float32)
    src = edge_index[0]
    dst = edge_index[1]
    batch_row = batch.reshape(1, n)
    zeros = jnp.zeros((n, d), jnp.float32)
    row = lambda a: a.reshape(1, -1)

    x0 = pl.pallas_call(
        _lin0_body,
        out_shape=jax.ShapeDtypeStruct((n, d), jnp.float32),
    )(x, W_lin, row(b_lin), row(bn0_g), row(bn0_b))

    segsum = _make_segsum(n, d, e)

    layer_call = pl.pallas_call(
        _layer_body,
        out_shape=(jax.ShapeDtypeStruct((n, d), jnp.float32),
                   jax.ShapeDtypeStruct((_G, d), jnp.float32)),
    )

    parts1 = segsum(x0, src, dst, zeros)
    h1, p1 = layer_call(x0, parts1, c1_W1, row(c1_b1), row(c1_g), row(c1_bb),
                        c1_W2, row(c1_b2), batch_row)
    parts2 = segsum(h1, src, dst, zeros)
    h2, p2 = layer_call(h1, parts2, c2_W1, row(c2_b1), row(c2_g), row(c2_bb),
                        c2_W2, row(c2_b2), batch_row)
    parts3 = segsum(h2, src, dst, zeros)
    out = pl.pallas_call(
        _layer3_head_body,
        out_shape=jax.ShapeDtypeStruct((_G, c), jnp.float32),
    )(h2, parts3, c3_W1, row(c3_b1), row(c3_g), row(c3_bb), c3_W2,
      row(c3_b2), batch_row, p1, p2, lin1_W, row(lin1_b), lin2_W,
      row(lin2_b))
    return out
